# baseline (speedup 1.0000x reference)
import functools

import jax
import jax.numpy as jnp
from jax import lax
from jax.experimental import pallas as pl
from jax.experimental.pallas import tpu as pltpu
from jax.experimental.pallas import tpu_sc as plsc

_D = 64
_SCALE = float(_D) ** 0.5
_NW = 32
_CHUNK = 128
_LANES = 16


def _make_kernel(B: int):
    bpw = B // _NW
    nchunk = bpw // _CHUNK
    mesh = plsc.VectorSubcoreMesh(core_axis_name="c", subcore_axis_name="s")

    @functools.partial(
        pl.kernel,
        mesh=mesh,
        out_type=jax.ShapeDtypeStruct((B, _D), jnp.float32),
        scratch_types=[
            pltpu.VMEM((nchunk, _CHUNK), jnp.int32),
            pltpu.VMEM((_CHUNK, _D), jnp.float32),
            pltpu.SemaphoreType.DMA,
        ],
        compiler_params=pltpu.CompilerParams(use_tc_tiling_on_sc=False),
    )
    def emb(idx_hbm, table_hbm, out_hbm, idx_v, rows_v, gsem):
        wid = lax.axis_index("s") * 2 + lax.axis_index("c")
        base = wid * bpw
        pltpu.sync_copy(idx_hbm.at[pl.ds(wid * nchunk, nchunk)], idx_v)

        def chunk_body(g, carry):
            pltpu.async_copy(table_hbm.at[idx_v.at[g]], rows_v, gsem).wait()

            def scale_body(r8, c2):
                for rr in range(8):
                    row = r8 * 8 + rr
                    for j in range(_D // _LANES):
                        sl = pl.ds(j * _LANES, _LANES)
                        rows_v[row, sl] = rows_v[row, sl] * _SCALE
                return c2

            lax.fori_loop(0, _CHUNK // 8, scale_body, 0)
            pltpu.sync_copy(rows_v, out_hbm.at[pl.ds(base + g * _CHUNK, _CHUNK)])
            return carry

        lax.fori_loop(0, nchunk, chunk_body, 0)

    return emb


@jax.jit
def kernel(x, table):
    B = x.shape[0] * x.shape[1]
    idx = x.reshape(B // _CHUNK, _CHUNK)
    out = _make_kernel(B)(idx, table)
    return out.reshape(x.shape[0], x.shape[1], _D)
